# initial kernel scaffold (unmeasured)
import jax
import jax.numpy as jnp
from jax import lax
from jax.experimental import pallas as pl
from jax.experimental.pallas import tpu as pltpu


def kernel(
    x,
):
    def body(*refs):
        pass

    out_shape = jax.ShapeDtypeStruct(..., jnp.float32)
    return pl.pallas_call(body, out_shape=out_shape)(...)


# baseline (device time: 49257 ns/iter reference)
import jax
import jax.numpy as jnp
from jax import lax
from jax.experimental import pallas as pl
from jax.experimental.pallas import tpu as pltpu

N_DEV = 16


def kernel(x):
    m, n = x.shape

    def body(x_ref, out_ref, stats_ref, comm_ref, send_sems, recv_sems):
        my = lax.axis_index("i")

        barrier_sem = pltpu.get_barrier_semaphore()
        for d in range(1, N_DEV):
            t = lax.rem(my + d, N_DEV)
            pl.semaphore_signal(
                barrier_sem,
                inc=1,
                device_id=(t,),
                device_id_type=pl.DeviceIdType.MESH,
            )
        pl.semaphore_wait(barrier_sem, N_DEV - 1)

        xv = x_ref[:, :]
        xm = jnp.max(xv, axis=1, keepdims=True)
        e = jnp.exp(xv - xm)
        s = jnp.sum(e, axis=1, keepdims=True)
        stats_ref[:, 0:1] = xm
        stats_ref[:, 1:2] = s

        rdmas = []
        for d in range(1, N_DEV):
            t = lax.rem(my + d, N_DEV)
            rdma = pltpu.make_async_remote_copy(
                src_ref=stats_ref,
                dst_ref=comm_ref.at[d - 1],
                send_sem=send_sems.at[d - 1],
                recv_sem=recv_sems.at[d - 1],
                device_id=(t,),
                device_id_type=pl.DeviceIdType.MESH,
            )
            rdma.start()
            rdmas.append(rdma)
        for rdma in rdmas:
            rdma.wait()

        gmax = xm
        for k in range(N_DEV - 1):
            gmax = jnp.maximum(gmax, comm_ref[k, :, 0:1])
        gsum = s * jnp.exp(xm - gmax)
        for k in range(N_DEV - 1):
            gsum = gsum + comm_ref[k, :, 1:2] * jnp.exp(
                comm_ref[k, :, 0:1] - gmax
            )

        out_ref[:, :] = e * (jnp.exp(xm - gmax) / gsum)

    return pl.pallas_call(
        body,
        out_shape=jax.ShapeDtypeStruct((m, n), jnp.float32),
        in_specs=[pl.BlockSpec(memory_space=pltpu.VMEM)],
        out_specs=pl.BlockSpec(memory_space=pltpu.VMEM),
        scratch_shapes=[
            pltpu.VMEM((m, 2), jnp.float32),
            pltpu.VMEM((N_DEV - 1, m, 2), jnp.float32),
            pltpu.SemaphoreType.DMA((N_DEV - 1,)),
            pltpu.SemaphoreType.DMA((N_DEV - 1,)),
        ],
        compiler_params=pltpu.CompilerParams(collective_id=0),
    )(x)


# device time: 10928 ns/iter; 4.5074x vs baseline; 4.5074x over previous
import jax
import jax.numpy as jnp
from jax import lax
from jax.experimental import pallas as pl
from jax.experimental.pallas import tpu as pltpu

N_DEV = 16


def kernel(x):
    m, n = x.shape

    def body(x_ref, out_ref, stats_ref, comm_ref, send_sems, recv_sems):
        my = lax.axis_index("i")

        barrier_sem = pltpu.get_barrier_semaphore()
        for d in range(1, N_DEV):
            t = lax.rem(my + d, N_DEV)
            pl.semaphore_signal(
                barrier_sem,
                inc=1,
                device_id=(t,),
                device_id_type=pl.DeviceIdType.MESH,
            )
        pl.semaphore_wait(barrier_sem, N_DEV - 1)

        xv = x_ref[:, :]
        xm = jnp.max(xv, axis=1, keepdims=True)
        e = jnp.exp(xv - xm)
        s = jnp.sum(e, axis=1, keepdims=True)
        stats_ref[:, :] = jnp.concatenate([xm, s], axis=1).T

        rdmas = []
        for d in range(1, N_DEV):
            t = lax.rem(my + d, N_DEV)
            rdma = pltpu.make_async_remote_copy(
                src_ref=stats_ref,
                dst_ref=comm_ref.at[d - 1],
                send_sem=send_sems.at[d - 1],
                recv_sem=recv_sems.at[d - 1],
                device_id=(t,),
                device_id_type=pl.DeviceIdType.MESH,
            )
            rdma.start()
            rdmas.append(rdma)
        for rdma in rdmas:
            rdma.wait()

        gmax_r = stats_ref[0:1, :]
        for k in range(N_DEV - 1):
            gmax_r = jnp.maximum(gmax_r, comm_ref[k, 0:1, :])
        gsum_r = stats_ref[1:2, :] * jnp.exp(stats_ref[0:1, :] - gmax_r)
        for k in range(N_DEV - 1):
            gsum_r = gsum_r + comm_ref[k, 1:2, :] * jnp.exp(
                comm_ref[k, 0:1, :] - gmax_r
            )

        gms = jnp.concatenate([gmax_r, gsum_r], axis=0).T
        gmax = gms[:, 0:1]
        gsum = gms[:, 1:2]

        out_ref[:, :] = e * (jnp.exp(xm - gmax) / gsum)

    return pl.pallas_call(
        body,
        out_shape=jax.ShapeDtypeStruct((m, n), jnp.float32),
        in_specs=[pl.BlockSpec(memory_space=pltpu.VMEM)],
        out_specs=pl.BlockSpec(memory_space=pltpu.VMEM),
        scratch_shapes=[
            pltpu.VMEM((2, m), jnp.float32),
            pltpu.VMEM((N_DEV - 1, 2, m), jnp.float32),
            pltpu.SemaphoreType.DMA((N_DEV - 1,)),
            pltpu.SemaphoreType.DMA((N_DEV - 1,)),
        ],
        compiler_params=pltpu.CompilerParams(collective_id=0),
    )(x)


# device time: 10631 ns/iter; 4.6333x vs baseline; 1.0279x over previous
import jax
import jax.numpy as jnp
from jax import lax
from jax.experimental import pallas as pl
from jax.experimental.pallas import tpu as pltpu

N_DEV = 16


def kernel(x):
    m, n = x.shape

    def body(x_ref, out_ref, stats_ref, comm_ref, send_sems, recv_sems):
        my = lax.axis_index("i")

        barrier_sem = pltpu.get_barrier_semaphore()
        for d in range(1, N_DEV):
            t = lax.rem(my + d, N_DEV)
            pl.semaphore_signal(
                barrier_sem,
                inc=1,
                device_id=(t,),
                device_id_type=pl.DeviceIdType.MESH,
            )

        xv = x_ref[:, :]
        xm = jnp.max(xv, axis=1, keepdims=True)
        e = jnp.exp(xv - xm)
        s = jnp.sum(e, axis=1, keepdims=True)
        stats_ref[:, :] = jnp.concatenate([xm, s], axis=1).T

        pl.semaphore_wait(barrier_sem, N_DEV - 1)

        rdmas = []
        for d in range(1, N_DEV):
            t = lax.rem(my + d, N_DEV)
            rdma = pltpu.make_async_remote_copy(
                src_ref=stats_ref,
                dst_ref=comm_ref.at[d - 1],
                send_sem=send_sems.at[d - 1],
                recv_sem=recv_sems.at[d - 1],
                device_id=(t,),
                device_id_type=pl.DeviceIdType.MESH,
            )
            rdma.start()
            rdmas.append(rdma)
        for rdma in rdmas:
            rdma.wait()

        peer_m = comm_ref[:, 0, :]
        peer_s = comm_ref[:, 1, :]
        gmax_r = jnp.maximum(
            jnp.max(peer_m, axis=0, keepdims=True), stats_ref[0:1, :]
        )
        gsum_r = stats_ref[1:2, :] * jnp.exp(stats_ref[0:1, :] - gmax_r)
        gsum_r = gsum_r + jnp.sum(
            peer_s * jnp.exp(peer_m - gmax_r), axis=0, keepdims=True
        )

        gms = jnp.concatenate([gmax_r, gsum_r], axis=0).T
        gmax = gms[:, 0:1]
        gsum = gms[:, 1:2]

        out_ref[:, :] = e * (jnp.exp(xm - gmax) / gsum)

    return pl.pallas_call(
        body,
        out_shape=jax.ShapeDtypeStruct((m, n), jnp.float32),
        in_specs=[pl.BlockSpec(memory_space=pltpu.VMEM)],
        out_specs=pl.BlockSpec(memory_space=pltpu.VMEM),
        scratch_shapes=[
            pltpu.VMEM((2, m), jnp.float32),
            pltpu.VMEM((N_DEV - 1, 2, m), jnp.float32),
            pltpu.SemaphoreType.DMA((N_DEV - 1,)),
            pltpu.SemaphoreType.DMA((N_DEV - 1,)),
        ],
        compiler_params=pltpu.CompilerParams(collective_id=0),
    )(x)


# device time: 10614 ns/iter; 4.6408x vs baseline; 1.0016x over previous
import jax
import jax.numpy as jnp
from jax import lax
from jax.experimental import pallas as pl
from jax.experimental.pallas import tpu as pltpu

N_DEV = 16


def kernel(x):
    m, n = x.shape

    def body(x_ref, out_ref, stats_ref, comm_ref, send_sems, recv_sems):
        my = lax.axis_index("i")

        barrier_sem = pltpu.get_barrier_semaphore()
        for d in range(1, N_DEV):
            t = lax.rem(my + d, N_DEV)
            pl.semaphore_signal(
                barrier_sem,
                inc=1,
                device_id=(t,),
                device_id_type=pl.DeviceIdType.MESH,
            )

        xv = x_ref[:, :]
        xm = jnp.max(xv, axis=1, keepdims=True)
        e = jnp.exp(xv - xm)
        s = jnp.sum(e, axis=1, keepdims=True)
        st = jnp.concatenate([xm, s], axis=1).T
        stats_ref[:, :] = jnp.concatenate(
            [st[r // 4 : r // 4 + 1, (r % 4) * 128 : (r % 4 + 1) * 128]
             for r in range(8)],
            axis=0,
        )

        pl.semaphore_wait(barrier_sem, N_DEV - 1)

        rdmas = []
        for d in range(1, N_DEV):
            t = lax.rem(my + d, N_DEV)
            rdma = pltpu.make_async_remote_copy(
                src_ref=stats_ref,
                dst_ref=comm_ref.at[d - 1],
                send_sem=send_sems.at[d - 1],
                recv_sem=recv_sems.at[d - 1],
                device_id=(t,),
                device_id_type=pl.DeviceIdType.MESH,
            )
            rdma.start()
            rdmas.append(rdma)
        for rdma in rdmas:
            rdma.wait()

        peer_m = comm_ref[:, 0:4, :]
        peer_s = comm_ref[:, 4:8, :]
        g4 = jnp.maximum(jnp.max(peer_m, axis=0), stats_ref[0:4, :])
        gs4 = stats_ref[4:8, :] * jnp.exp(stats_ref[0:4, :] - g4)
        gs4 = gs4 + jnp.sum(peer_s * jnp.exp(peer_m - g4[None]), axis=0)

        gmax_r = jnp.concatenate([g4[i : i + 1, :] for i in range(4)], axis=1)
        gsum_r = jnp.concatenate([gs4[i : i + 1, :] for i in range(4)], axis=1)
        gms = jnp.concatenate([gmax_r, gsum_r], axis=0).T
        gmax = gms[:, 0:1]
        gsum = gms[:, 1:2]

        out_ref[:, :] = e * (jnp.exp(xm - gmax) / gsum)

    return pl.pallas_call(
        body,
        out_shape=jax.ShapeDtypeStruct((m, n), jnp.float32),
        in_specs=[pl.BlockSpec(memory_space=pltpu.VMEM)],
        out_specs=pl.BlockSpec(memory_space=pltpu.VMEM),
        scratch_shapes=[
            pltpu.VMEM((8, 128), jnp.float32),
            pltpu.VMEM((N_DEV - 1, 8, 128), jnp.float32),
            pltpu.SemaphoreType.DMA((N_DEV - 1,)),
            pltpu.SemaphoreType.DMA((N_DEV - 1,)),
        ],
        compiler_params=pltpu.CompilerParams(collective_id=0),
    )(x)


# device time: 10561 ns/iter; 4.6640x vs baseline; 1.0050x over previous
import jax
import jax.numpy as jnp
from jax import lax
from jax.experimental import pallas as pl
from jax.experimental.pallas import tpu as pltpu

N_DEV = 16


def kernel(x):
    m, n = x.shape

    def body(x_ref, out_ref, stats_ref, comm_ref, send_sems, recv_sems):
        my = lax.axis_index("i")

        barrier_sem = pltpu.get_barrier_semaphore()
        for d in range(1, N_DEV):
            t = lax.rem(my + d, N_DEV)
            pl.semaphore_signal(
                barrier_sem,
                inc=1,
                device_id=(t,),
                device_id_type=pl.DeviceIdType.MESH,
            )

        xv = x_ref[:, :]
        xm = jnp.max(xv, axis=1, keepdims=True)
        s = jnp.sum(jnp.exp(xv - xm), axis=1, keepdims=True)
        st = jnp.concatenate([xm, s], axis=1).T
        stats_ref[:, :] = jnp.concatenate(
            [st[r // 4 : r // 4 + 1, (r % 4) * 128 : (r % 4 + 1) * 128]
             for r in range(8)],
            axis=0,
        )

        pl.semaphore_wait(barrier_sem, N_DEV - 1)

        rdmas = []
        for d in range(1, N_DEV):
            t = lax.rem(my + d, N_DEV)
            rdma = pltpu.make_async_remote_copy(
                src_ref=stats_ref,
                dst_ref=comm_ref.at[d - 1],
                send_sem=send_sems.at[d - 1],
                recv_sem=recv_sems.at[d - 1],
                device_id=(t,),
                device_id_type=pl.DeviceIdType.MESH,
            )
            rdma.start()
            rdmas.append(rdma)

        e = jnp.exp(xv - xm)

        for rdma in rdmas:
            rdma.wait_recv()

        peer_m = comm_ref[:, 0:4, :]
        peer_s = comm_ref[:, 4:8, :]
        g4 = jnp.maximum(jnp.max(peer_m, axis=0), stats_ref[0:4, :])
        gs4 = stats_ref[4:8, :] * jnp.exp(stats_ref[0:4, :] - g4)
        gs4 = gs4 + jnp.sum(peer_s * jnp.exp(peer_m - g4[None]), axis=0)

        gmax_r = jnp.concatenate([g4[i : i + 1, :] for i in range(4)], axis=1)
        gsum_r = jnp.concatenate([gs4[i : i + 1, :] for i in range(4)], axis=1)
        gms = jnp.concatenate([gmax_r, gsum_r], axis=0).T
        gmax = gms[:, 0:1]
        gsum = gms[:, 1:2]

        out_ref[:, :] = e * (jnp.exp(xm - gmax) / gsum)

        for rdma in rdmas:
            rdma.wait_send()

    return pl.pallas_call(
        body,
        out_shape=jax.ShapeDtypeStruct((m, n), jnp.float32),
        in_specs=[pl.BlockSpec(memory_space=pltpu.VMEM)],
        out_specs=pl.BlockSpec(memory_space=pltpu.VMEM),
        scratch_shapes=[
            pltpu.VMEM((8, 128), jnp.float32),
            pltpu.VMEM((N_DEV - 1, 8, 128), jnp.float32),
            pltpu.SemaphoreType.DMA((N_DEV - 1,)),
            pltpu.SemaphoreType.DMA((N_DEV - 1,)),
        ],
        compiler_params=pltpu.CompilerParams(collective_id=0),
    )(x)
